# baseline (speedup 1.0000x reference)
import functools
import jax
import jax.numpy as jnp
from jax import lax
from jax.experimental import pallas as pl
from jax.experimental.pallas import tpu as pltpu
from jax.experimental.pallas import tpu_sc as plsc

D = 32
K = 8
ND = D * K
W = 2 * ND + K
N = 16384

NC = 2
NS = 16
NW = NC * NS
RPW = N // NW
CH = 64
NCH = RPW // CH


def _sc_body(x_hbm, mean_hbm, std_hbm, pi_hbm,
             sbuf, obuf, pibuf, pobuf, sem_mean, sem_pi):
    wid = lax.axis_index("s") * NC + lax.axis_index("c")
    base = wid * RPW

    mean_cp = pltpu.make_async_copy(
        x_hbm.at[pl.ds(base, RPW), pl.ds(0, ND)],
        mean_hbm.at[pl.ds(base, RPW)], sem_mean)
    mean_cp.start()

    pi_cp = pltpu.make_async_copy(
        x_hbm.at[pl.ds(base, RPW), pl.ds(2 * ND, K)], pibuf, sem_pi)
    pi_cp.start()

    def chunk(c, carry):
        r0 = base + c * CH
        pltpu.sync_copy(x_hbm.at[pl.ds(r0, CH), pl.ds(ND, ND)], sbuf)

        def row(r, carry2):
            for j in range(ND // 16):
                v = sbuf[r, pl.ds(j * 16, 16)]
                obuf[r, pl.ds(j * 16, 16)] = 1.0 / (1.0 + jnp.exp(-v))
            return carry2
        lax.fori_loop(0, CH, row, 0)
        pltpu.sync_copy(obuf, std_hbm.at[pl.ds(r0, CH)])
        return carry
    lax.fori_loop(0, NCH, chunk, 0)

    pi_cp.wait()
    lane = lax.iota(jnp.int32, 16)

    def pirow(g, carry):
        rows = g * 16 + lane
        cols = [plsc.load_gather(pibuf, [rows, jnp.full((16,), k, jnp.int32)])
                for k in range(K)]
        m = cols[0]
        for k in range(1, K):
            m = jnp.maximum(m, cols[k])
        es = [jnp.exp(c - m) for c in cols]
        s = es[0]
        for k in range(1, K):
            s = s + es[k]
        inv = 1.0 / s
        for k in range(K):
            plsc.store_scatter(pobuf, [rows, jnp.full((16,), k, jnp.int32)],
                               es[k] * inv)
        return carry
    lax.fori_loop(0, RPW // 16, pirow, 0)
    pltpu.sync_copy(pobuf, pi_hbm.at[pl.ds(base, RPW)])

    mean_cp.wait()


def _sc_call(x):
    f = pl.kernel(
        _sc_body,
        mesh=plsc.VectorSubcoreMesh(core_axis_name="c", subcore_axis_name="s"),
        out_type=[
            jax.ShapeDtypeStruct((N, ND), jnp.float32),
            jax.ShapeDtypeStruct((N, ND), jnp.float32),
            jax.ShapeDtypeStruct((N, K), jnp.float32),
        ],
        scratch_types=[
            pltpu.VMEM((CH, ND), jnp.float32),
            pltpu.VMEM((CH, ND), jnp.float32),
            pltpu.VMEM((RPW, K), jnp.float32),
            pltpu.VMEM((RPW, K), jnp.float32),
            pltpu.SemaphoreType.DMA,
            pltpu.SemaphoreType.DMA,
        ],
        compiler_params=pltpu.CompilerParams(
            needs_layout_passes=False, use_tc_tiling_on_sc=False),
    )
    return f(x)


def kernel(x):
    mean2d, std2d, pi = _sc_call(x)
    return (mean2d.reshape(N, D, K), std2d.reshape(N, D, K), pi)
